# baseline (speedup 1.0000x reference)
import functools

import jax
import jax.numpy as jnp
from jax import lax
from jax.experimental import pallas as pl
from jax.experimental.pallas import tpu as pltpu
from jax.experimental.pallas import tpu_sc as plsc

NUM_FRAME = 8
TOPK = 64
TOP_ATTN = 6
P = 196
D = 768
NHEAD = 12
NLAYER = 12
NMAPS = (NLAYER - TOP_ATTN) * NHEAD
HALF_ROWS = NMAPS // 2
NCHUNK = 13
SLAB_W = 208
COPY_W = 200
PSUM_W = NCHUNK * 16
BT = 2 * NUM_FRAME
HALF_K = TOPK // 2


def _body(tok_hbm, am_hbm, out_hbm, idx_hbm,
          slab, psum, pbuf, idxbuf, gbuf, idxv, rows,
          ps_sh, g_sh, sem):
    c = lax.axis_index("c")
    s = lax.axis_index("s")
    pair = s // 2
    half = s % 2
    bt = c * (BT // 2) + pair
    base_row = bt * (NLAYER * NHEAD) + TOP_ATTN * NHEAD + half * HALF_ROWS

    pltpu.sync_copy(am_hbm.at[pl.ds(base_row, HALF_ROWS), pl.ds(0, COPY_W)],
                    slab.at[:, pl.ds(0, COPY_W)])
    acc = [jnp.zeros((16,), jnp.float32) for _ in range(NCHUNK)]
    for j in range(HALF_ROWS):
        for q in range(NCHUNK):
            acc[q] = acc[q] + slab[j, pl.ds(16 * q, 16)]
    for q in range(NCHUNK):
        psum[pl.ds(16 * q, 16)] = acc[q]
    pltpu.sync_copy(psum, ps_sh.at[s])
    plsc.subcore_barrier()

    lane = lax.iota(jnp.int32, 16)

    @pl.when(half == 0)
    def _select():
        pltpu.sync_copy(ps_sh.at[s + 1], pbuf)
        sc = [acc[q] + pbuf[pl.ds(16 * q, 16)] for q in range(NCHUNK)]
        sc[0] = jnp.where(lane == 0, -jnp.inf, sc[0])
        sc[NCHUNK - 1] = jnp.where(lane <= P - 16 * (NCHUNK - 1),
                                   sc[NCHUNK - 1], -jnp.inf)
        gidx = [16 * q + lane for q in range(NCHUNK)]
        mask0 = lane == 0
        big = jnp.int32(1 << 30)

        def step(k, carry):
            svecs = list(carry)
            m = svecs[0]
            for q in range(1, NCHUNK):
                m = jnp.maximum(m, svecs[q])
            mmax = jnp.max(m)
            best = jnp.full((16,), big, jnp.int32)
            for q in range(NCHUNK):
                best = jnp.minimum(best,
                                   jnp.where(svecs[q] == mmax, gidx[q], big))
            mi = jnp.min(best)
            miv = jnp.full((16,), mi, jnp.int32)
            for q in range(NCHUNK):
                svecs[q] = jnp.where(gidx[q] == miv, -jnp.inf, svecs[q])
            plsc.store_scatter(idxbuf, [jnp.full((16,), k, jnp.int32)],
                               miv - 1, mask=mask0)
            return tuple(svecs)

        lax.fori_loop(0, TOPK, step, tuple(sc))

        b = bt // NUM_FRAME
        t = bt % NUM_FRAME
        row0 = b * (NUM_FRAME * P) + t * P
        for q in range(TOPK // 16):
            gbuf[pl.ds(16 * q, 16)] = idxbuf[pl.ds(16 * q, 16)] + row0
        pltpu.sync_copy(gbuf, g_sh.at[s])
        pltpu.sync_copy(idxbuf, idx_hbm.at[bt])

    plsc.subcore_barrier()

    pltpu.sync_copy(g_sh.at[pair * 2, pl.ds(half * HALF_K, HALF_K)], idxv)
    pltpu.async_copy(tok_hbm.at[idxv], rows, sem).wait()
    pltpu.sync_copy(rows,
                    out_hbm.at[pl.ds(bt * TOPK + half * HALF_K, HALF_K), :])


@jax.jit
def _run(tok, am):
    kfn = pl.kernel(
        _body,
        out_type=[
            jax.ShapeDtypeStruct((BT * TOPK, D), jnp.float32),
            jax.ShapeDtypeStruct((BT, TOPK), jnp.int32),
        ],
        mesh=plsc.VectorSubcoreMesh(core_axis_name="c", subcore_axis_name="s"),
        compiler_params=pltpu.CompilerParams(use_tc_tiling_on_sc=False,
                                             needs_layout_passes=False),
        scratch_types=[
            pltpu.VMEM((HALF_ROWS, SLAB_W), jnp.float32),
            pltpu.VMEM((PSUM_W,), jnp.float32),
            pltpu.VMEM((PSUM_W,), jnp.float32),
            pltpu.VMEM((TOPK,), jnp.int32),
            pltpu.VMEM((TOPK,), jnp.int32),
            pltpu.VMEM((HALF_K,), jnp.int32),
            pltpu.VMEM((HALF_K, D), jnp.float32),
            pltpu.VMEM_SHARED((16, PSUM_W), jnp.float32),
            pltpu.VMEM_SHARED((16, TOPK), jnp.int32),
            pltpu.SemaphoreType.DMA,
        ],
    )
    return kfn(tok, am)


def kernel(tokens, attn_maps):
    B = tokens.shape[0]
    tok = tokens.reshape(B * NUM_FRAME * P, D)
    am = attn_maps.reshape(B * NUM_FRAME * NLAYER * NHEAD, (P + 1) * (P + 1))
    out, idx = _run(tok, am)
    return (out.reshape(B, NUM_FRAME * TOPK, D),
            idx.reshape(B, NUM_FRAME, TOPK))


# speedup vs baseline: 34.8205x; 34.8205x over previous
import functools

import jax
import jax.numpy as jnp
from jax import lax
from jax.experimental import pallas as pl
from jax.experimental.pallas import tpu as pltpu
from jax.experimental.pallas import tpu_sc as plsc

NUM_FRAME = 8
TOPK = 64
TOP_ATTN = 6
P = 196
D = 768
NHEAD = 12
NLAYER = 12
NMAPS = (NLAYER - TOP_ATTN) * NHEAD
HALF_ROWS = NMAPS // 2
NCHUNK = 13
SLAB_W = 208
COPY_W = 200
PSUM_W = NCHUNK * 16
BT = 2 * NUM_FRAME
HALF_K = TOPK // 2


def _body(tok_hbm, am_hbm, out_hbm, idx_hbm,
          slab, psum, pbuf, idxbuf, gbuf, idxv, rows,
          ps_sh, g_sh, sem):
    c = lax.axis_index("c")
    s = lax.axis_index("s")
    pair = s // 2
    half = s % 2
    bt = c * (BT // 2) + pair
    b = bt // NUM_FRAME
    t = bt % NUM_FRAME
    base_row = bt * NMAPS + half * HALF_ROWS

    pltpu.sync_copy(am_hbm.at[pl.ds(base_row, HALF_ROWS)], slab)
    acc = [jnp.zeros((16,), jnp.float32) for _ in range(NCHUNK)]
    for j in range(HALF_ROWS):
        for q in range(NCHUNK):
            acc[q] = acc[q] + slab[j, pl.ds(16 * q, 16)]
    for q in range(NCHUNK):
        psum[pl.ds(16 * q, 16)] = acc[q]
    pltpu.sync_copy(psum, ps_sh.at[s])
    plsc.subcore_barrier()

    lane = lax.iota(jnp.int32, 16)

    @pl.when(half == 0)
    def _select():
        pltpu.sync_copy(ps_sh.at[s + 1], pbuf)
        sc = [acc[q] + pbuf[pl.ds(16 * q, 16)] for q in range(NCHUNK)]
        sc[NCHUNK - 1] = jnp.where(lane < P - 16 * (NCHUNK - 1),
                                   sc[NCHUNK - 1], -jnp.inf)
        gidx = [16 * q + lane for q in range(NCHUNK)]
        mask0 = lane == 0
        big = jnp.int32(1 << 30)

        def step(k, carry):
            svecs = list(carry)
            m = svecs[0]
            for q in range(1, NCHUNK):
                m = jnp.maximum(m, svecs[q])
            mmax = jnp.max(m)
            best = jnp.full((16,), big, jnp.int32)
            for q in range(NCHUNK):
                best = jnp.minimum(best,
                                   jnp.where(svecs[q] == mmax, gidx[q], big))
            mi = jnp.min(best)
            miv = jnp.full((16,), mi, jnp.int32)
            for q in range(NCHUNK):
                svecs[q] = jnp.where(gidx[q] == miv, -jnp.inf, svecs[q])
            plsc.store_scatter(idxbuf, [jnp.full((16,), k, jnp.int32)],
                               miv, mask=mask0)
            return tuple(svecs)

        lax.fori_loop(0, TOPK, step, tuple(sc))

        row0 = b * (NUM_FRAME * P) + t * P
        for q in range(TOPK // 16):
            gbuf[pl.ds(16 * q, 16)] = idxbuf[pl.ds(16 * q, 16)] + row0
        pltpu.sync_copy(gbuf, g_sh.at[s])
        pltpu.sync_copy(idxbuf, idx_hbm.at[b, t])

    plsc.subcore_barrier()

    pltpu.sync_copy(g_sh.at[pair * 2, pl.ds(half * HALF_K, HALF_K)], idxv)
    pltpu.async_copy(tok_hbm.at[idxv], rows, sem).wait()
    pltpu.sync_copy(
        rows, out_hbm.at[b, pl.ds(t * TOPK + half * HALF_K, HALF_K), :])


@jax.jit
def _run(tok, am):
    kfn = pl.kernel(
        _body,
        out_type=[
            jax.ShapeDtypeStruct((2, NUM_FRAME * TOPK, D), jnp.float32),
            jax.ShapeDtypeStruct((2, NUM_FRAME, TOPK), jnp.int32),
        ],
        mesh=plsc.VectorSubcoreMesh(core_axis_name="c", subcore_axis_name="s"),
        compiler_params=pltpu.CompilerParams(use_tc_tiling_on_sc=False,
                                             needs_layout_passes=False),
        scratch_types=[
            pltpu.VMEM((HALF_ROWS, SLAB_W), jnp.float32),
            pltpu.VMEM((PSUM_W,), jnp.float32),
            pltpu.VMEM((PSUM_W,), jnp.float32),
            pltpu.VMEM((TOPK,), jnp.int32),
            pltpu.VMEM((TOPK,), jnp.int32),
            pltpu.VMEM((HALF_K,), jnp.int32),
            pltpu.VMEM((HALF_K, D), jnp.float32),
            pltpu.VMEM_SHARED((16, PSUM_W), jnp.float32),
            pltpu.VMEM_SHARED((16, TOPK), jnp.int32),
            pltpu.SemaphoreType.DMA,
        ],
    )
    return kfn(tok, am)


def kernel(tokens, attn_maps):
    B = tokens.shape[0]
    tok = tokens.reshape(B * NUM_FRAME * P, D)
    am = attn_maps[:, :, TOP_ATTN:, :, 0, 1:]
    am = am.reshape(B * NUM_FRAME * NMAPS, P)
    am = jnp.pad(am, ((0, 0), (0, SLAB_W - P)))
    out, idx = _run(tok, am)
    return out, idx
